# Initial kernel scaffold; baseline (speedup 1.0000x reference)
import jax
import jax.numpy as jnp
from jax.experimental import pallas as pl


def kernel(x, y):
    raise NotImplementedError("write your pallas kernel here")


# speedup vs baseline: 59.4198x; 59.4198x over previous
import functools

import jax
import jax.numpy as jnp
from jax import lax
from jax.experimental import pallas as pl
from jax.experimental.pallas import tpu as pltpu
from jax.experimental.pallas import tpu_sc as plsc

B = 128
N = 100000
KTH = 10
L = 16
NW = 32
ROWS_PER_W = B // NW
GRP_V = 50
NGRP = N // (L * GRP_V)
NEG = float("-inf")

_mesh = plsc.VectorSubcoreMesh(core_axis_name="c", subcore_axis_name="s")
_cparams = pltpu.CompilerParams(needs_layout_passes=False)


def _merge_top16(best_asc, vec):
    v_desc = lax.rev(lax.sort(vec), (0,))
    return lax.sort(jnp.maximum(best_asc, v_desc))


@functools.partial(
    pl.kernel,
    out_type=[
        jax.ShapeDtypeStruct((NW, L), jnp.float32),
        jax.ShapeDtypeStruct((NW, L), jnp.float32),
    ],
    mesh=_mesh,
    compiler_params=_cparams,
    scratch_types=[
        pltpu.VMEM((N,), jnp.float32),
        pltpu.VMEM((NGRP * L,), jnp.float32),
        pltpu.VMEM((B,), jnp.int32),
        pltpu.VMEM((L,), jnp.float32),
        pltpu.VMEM((L,), jnp.float32),
    ],
)
def _topk_gather(x_hbm, y_hbm, stopk_hbm, sy_hbm, row_v, summ_v, y_v, tk_v, sy_v):
    wid = lax.axis_index("s") * 2 + lax.axis_index("c")
    pltpu.sync_copy(y_hbm, y_v)
    iota = lax.iota(jnp.int32, L)
    tk_res = jnp.full((L,), NEG, jnp.float32)
    sy_res = jnp.full((L,), NEG, jnp.float32)

    for r in range(ROWS_PER_W):
        row = wid * ROWS_PER_W + r
        pltpu.sync_copy(x_hbm.at[row], row_v)

        def grp_body(gi, best):
            m = row_v[pl.ds(gi * (GRP_V * L), L)]
            for j in range(1, GRP_V):
                m = jnp.maximum(m, row_v[pl.ds(gi * (GRP_V * L) + j * L, L)])
            summ_v[pl.ds(gi * L, L)] = m
            return _merge_top16(best, m)

        best = lax.fori_loop(0, NGRP, grp_body,
                             jnp.full((L,), NEG, jnp.float32))
        t = jnp.max(jnp.where(iota == (L - 1 - KTH), best, NEG))

        def p2_body(gi, carry):
            cnt, best2 = carry
            sm = summ_v[pl.ds(gi * L, L)]

            def dig(c):
                cnt2, b2 = c
                for j in range(GRP_V):
                    v = row_v[pl.ds(gi * (GRP_V * L) + j * L, L)]
                    msk = v > t
                    cnt2 = cnt2 + jnp.sum(
                        jnp.where(msk, jnp.int32(1), jnp.int32(0)))
                    b2 = _merge_top16(b2, jnp.where(msk, v, NEG))
                return cnt2, b2

            return lax.cond(jnp.max(sm) > t, dig, lambda c: c, carry)

        cnt, best2 = lax.fori_loop(
            0, NGRP, p2_body,
            (jnp.int32(0), jnp.full((L,), NEG, jnp.float32)))
        e11 = jnp.max(jnp.where(iota == (L - 1 - KTH), best2, NEG))
        ans = jnp.where(cnt <= KTH, t, e11)
        tk_res = jnp.where(iota == r, ans, tk_res)

        yvec = y_v[pl.ds((row // L) * L, L)]
        yi = jnp.max(jnp.where(iota == row % L, yvec, jnp.int32(-1)))
        q = yi // L
        lane = yi - q * L
        v = row_v[pl.ds(q * L, L)]
        sel = jnp.max(jnp.where(iota == lane, v, NEG))
        sy_res = jnp.where(iota == r, sel, sy_res)

    tk_v[...] = tk_res
    sy_v[...] = sy_res
    pltpu.sync_copy(tk_v, stopk_hbm.at[wid])
    pltpu.sync_copy(sy_v, sy_hbm.at[wid])


@functools.partial(
    pl.kernel,
    out_type=jax.ShapeDtypeStruct((L,), jnp.float32),
    mesh=_mesh,
    compiler_params=_cparams,
    scratch_types=[
        pltpu.VMEM((NW, L), jnp.float32),
        pltpu.VMEM((NW, L), jnp.float32),
        pltpu.VMEM((L,), jnp.float32),
    ],
)
def _pair_mean(stopk_hbm, sy_hbm, out_hbm, tk_v, sy_v, o_v):
    wid = lax.axis_index("s") * 2 + lax.axis_index("c")

    @pl.when(wid == 0)
    def _():
        pltpu.sync_copy(stopk_hbm, tk_v)
        pltpu.sync_copy(sy_hbm, sy_v)
        tvs = [1.0 + tk_v[w] for w in range(NW)]

        iota = lax.iota(jnp.int32, L)

        def i_body(i, acc):
            svec = sy_v[i // ROWS_PER_W]
            syi = jnp.max(jnp.where(iota == i % ROWS_PER_W, svec, NEG))
            for w in range(NW):
                acc = acc + jnp.maximum(tvs[w] - syi, 0.0)
            return acc

        acc = lax.fori_loop(0, B, i_body, jnp.zeros((L,), jnp.float32))
        total = jnp.sum(acc)
        o_v[...] = jnp.full((L,), total * (1.0 / (B * B)), jnp.float32)
        pltpu.sync_copy(o_v, out_hbm)


def kernel(x, y):
    stopk, sy = _topk_gather(x, y.astype(jnp.int32))
    out = _pair_mean(stopk, sy)
    return out[0]
